# baseline (speedup 1.0000x reference)
import jax
import jax.numpy as jnp
import numpy as np
from jax.experimental import pallas as pl

N = 50000
E = 800000
H = 8
C = 16
D = 4
KEY_FEATS = 64


def _copy_body(x_ref, o_ref):
    o_ref[...] = x_ref[...]


def kernel(value, key, query_0, query_1, edge_index):
    dst = edge_index[1].astype(jnp.int32)
    k = key.reshape(E, H, -1)
    q = jnp.concatenate([query_0, query_1], axis=-1).reshape(N, H, -1)
    q_dst = jnp.take(q, dst, axis=0)
    logits = jnp.sum(k * q_dst, axis=-1) / np.sqrt(KEY_FEATS)
    seg_max = jax.ops.segment_max(logits, dst, num_segments=N)
    seg_max = jnp.where(jnp.isfinite(seg_max), seg_max, 0.0)
    expw = jnp.exp(logits - jnp.take(seg_max, dst, axis=0))
    denom = jax.ops.segment_sum(expw, dst, num_segments=N)
    w = expw / (jnp.take(denom, dst, axis=0) + 1e-9)
    v = value.reshape(E, H, C // H, D)
    feat = jax.ops.segment_sum(w[..., None, None] * v, dst, num_segments=N)
    feat = feat.reshape(N // 2, 2 * C * D)
    feat = pl.pallas_call(
        _copy_body,
        grid=(5,),
        in_specs=[pl.BlockSpec((N // 10, 2 * C * D), lambda i: (i, 0))],
        out_specs=pl.BlockSpec((N // 10, 2 * C * D), lambda i: (i, 0)),
        out_shape=jax.ShapeDtypeStruct((N // 2, 2 * C * D), jnp.float32),
    )(feat)
    feat = feat.reshape(N, C, D)
    return feat[..., :1], feat[..., 1:4]


# speedup vs baseline: 17.4431x; 17.4431x over previous
import jax
import jax.numpy as jnp
from jax.experimental import pallas as pl

N = 50000
E = 800000
H = 8
F = 8
BE = 8000
GRID = E // BE


def _edge_body(k_ref, q_ref, v_ref, wv_ref, w_ref):
    from jax import lax
    k = k_ref[...]
    q = q_ref[...]
    v = v_ref[...]
    prod = k * q
    r = lax.broadcasted_iota(jnp.int32, (H * F, H * F), 0) // F
    cc = lax.broadcasted_iota(jnp.int32, (H * F, H * F), 1) // F
    a = (r == cc).astype(jnp.float32)
    logits = jnp.dot(prod, a, preferred_element_type=jnp.float32)
    w = jnp.exp(logits * 0.125)
    wv_ref[...] = w * v
    w_ref[...] = w


def kernel(value, key, query_0, query_1, edge_index):
    q = jnp.concatenate([query_0, query_1], axis=-1).reshape(N, H * F)
    v = value.reshape(E, H * F)
    dst = edge_index[1].astype(jnp.int32)
    q_dst = jnp.take(q, dst, axis=0)

    wv, w = pl.pallas_call(
        _edge_body,
        grid=(GRID,),
        in_specs=[
            pl.BlockSpec((BE, H * F), lambda i: (i, 0)),
            pl.BlockSpec((BE, H * F), lambda i: (i, 0)),
            pl.BlockSpec((BE, H * F), lambda i: (i, 0)),
        ],
        out_specs=(
            pl.BlockSpec((BE, H * F), lambda i: (i, 0)),
            pl.BlockSpec((BE, H * F), lambda i: (i, 0)),
        ),
        out_shape=(
            jax.ShapeDtypeStruct((E, H * F), jnp.float32),
            jax.ShapeDtypeStruct((E, H * F), jnp.float32),
        ),
    )(key, q_dst, v)

    num = jax.ops.segment_sum(wv, dst, num_segments=N)
    den = jax.ops.segment_sum(w, dst, num_segments=N)
    feat = (num / jnp.maximum(den, 1e-30)).reshape(N, 16, 4)
    return feat[..., :1], feat[..., 1:4]
